# baseline (speedup 1.0000x reference)
import jax
import jax.numpy as jnp
from jax.experimental import pallas as pl

N = 10000
E = 160000
D_FEAT = 256
D_EDGE = 16
D_OUT = 256
EDGE_BLK = 8000


def _node_mm_kernel(x_ref, w_ref, b_ref, o_ref):
    o_ref[...] = (
        jnp.dot(x_ref[...], w_ref[...], preferred_element_type=jnp.float32)
        + b_ref[...]
    )


def _edge_msg_kernel(tns_ref, ea_ref, we_ref, be_ref, norm_ref, o_ref):
    edges = (
        jnp.dot(ea_ref[...], we_ref[...], preferred_element_type=jnp.float32)
        + be_ref[...]
    )
    o_ref[...] = norm_ref[...] * jnp.maximum(tns_ref[...] + edges, 0.0)


def _combine_kernel(agg_ref, tn_ref, root_ref, invdeg_ref, o_ref):
    self_msg = jnp.maximum(tn_ref[...] + root_ref[...], 0.0) * invdeg_ref[...]
    o_ref[...] = jnp.maximum(agg_ref[...] + self_msg, 0.0)


def kernel(x, edge_attr, senders, receivers, W_node, b_node, W_edge, b_edge, root_emb):
    transf = pl.pallas_call(
        _node_mm_kernel,
        out_shape=jax.ShapeDtypeStruct((N, D_OUT), jnp.float32),
    )(x, W_node, b_node.reshape(1, D_OUT))

    ones = jnp.ones((E,), jnp.float32)
    sender_degree = jax.ops.segment_sum(ones, senders, num_segments=N) + 1.0
    receiver_degree = jax.ops.segment_sum(ones, receivers, num_segments=N) + 1.0
    norm = (
        jax.lax.rsqrt(sender_degree)[senders]
        * jax.lax.rsqrt(receiver_degree)[receivers]
    )[:, None]

    tn_s = transf[senders]

    messages = pl.pallas_call(
        _edge_msg_kernel,
        grid=(E // EDGE_BLK,),
        in_specs=[
            pl.BlockSpec((EDGE_BLK, D_OUT), lambda i: (i, 0)),
            pl.BlockSpec((EDGE_BLK, D_EDGE), lambda i: (i, 0)),
            pl.BlockSpec((D_EDGE, D_OUT), lambda i: (0, 0)),
            pl.BlockSpec((1, D_OUT), lambda i: (0, 0)),
            pl.BlockSpec((EDGE_BLK, 1), lambda i: (i, 0)),
        ],
        out_specs=pl.BlockSpec((EDGE_BLK, D_OUT), lambda i: (i, 0)),
        out_shape=jax.ShapeDtypeStruct((E, D_OUT), jnp.float32),
    )(tn_s, edge_attr, W_edge, b_edge.reshape(1, D_OUT), norm)

    agg = jax.ops.segment_sum(messages, receivers, num_segments=N)

    out = pl.pallas_call(
        _combine_kernel,
        out_shape=jax.ShapeDtypeStruct((N, D_OUT), jnp.float32),
    )(agg, transf, root_emb, (1.0 / receiver_degree)[:, None])

    return out
